# baseline (speedup 1.0000x reference)
import jax
import jax.numpy as jnp
from jax import lax
from jax.experimental import pallas as pl
from jax.experimental.pallas import tpu as pltpu
from jax.experimental.pallas import tpu_sc as plsc

_CR = 32


def _make_sc_kernel(B, S, H, dtype):
    info = plsc.get_sparse_core_info()
    NC, NS = info.num_cores, info.num_subcores
    NW = NC * NS
    rows_per_w = S // NW
    nchunk = rows_per_w // _CR
    mesh = plsc.VectorSubcoreMesh(core_axis_name="c", subcore_axis_name="s")

    def body(pos_hbm, out_hbm, vbuf, in_sem, out_sem):
        wid = lax.axis_index("s") * NC + lax.axis_index("c")
        base = wid * rows_per_w

        def in_cp(k, slot):
            return pltpu.make_async_copy(
                pos_hbm.at[pl.ds(base + k * _CR, _CR), :],
                vbuf.at[slot],
                in_sem.at[slot],
            )

        def out_cp(k, slot, b):
            return pltpu.make_async_copy(
                vbuf.at[slot],
                out_hbm.at[b, pl.ds(base + k * _CR, _CR), :],
                out_sem.at[slot],
            )

        in_cp(0, 0).start()
        for k in range(nchunk):
            slot = k % 2
            in_cp(k, slot).wait()
            if k + 1 < nchunk:
                if k >= 1:
                    for b in range(B):
                        out_cp(k - 1, 1 - slot, b).wait()
                in_cp(k + 1, 1 - slot).start()
            for b in range(B):
                out_cp(k, slot, b).start()
        for k in (nchunk - 2, nchunk - 1):
            for b in range(B):
                out_cp(k, k % 2, b).wait()

    return pl.kernel(
        body,
        out_type=jax.ShapeDtypeStruct((B, S, H), dtype),
        mesh=mesh,
        scratch_types=[
            pltpu.VMEM((2, _CR, H), dtype),
            pltpu.SemaphoreType.DMA((2,)),
            pltpu.SemaphoreType.DMA((2,)),
        ],
    )


def kernel(x, pos_emb):
    B, S = x.shape
    N, H = pos_emb.shape
    return _make_sc_kernel(B, S, H, pos_emb.dtype)(pos_emb)


# speedup vs baseline: 1.0064x; 1.0064x over previous
import jax
import jax.numpy as jnp
from jax import lax
from jax.experimental import pallas as pl
from jax.experimental.pallas import tpu as pltpu
from jax.experimental.pallas import tpu_sc as plsc

_CR = 32
_RING = 3


def _make_sc_kernel(B, S, H, dtype):
    info = plsc.get_sparse_core_info()
    NC, NS = info.num_cores, info.num_subcores
    NW = NC * NS
    rows_per_w = S // NW
    nchunk = rows_per_w // _CR
    mesh = plsc.VectorSubcoreMesh(core_axis_name="c", subcore_axis_name="s")

    def body(pos_hbm, out_hbm, vbuf, in_sem, out_sem):
        wid = lax.axis_index("s") * NC + lax.axis_index("c")
        base = wid * rows_per_w

        def in_cp(k, slot):
            return pltpu.make_async_copy(
                pos_hbm.at[pl.ds(base + k * _CR, _CR), :],
                vbuf.at[slot],
                in_sem.at[slot],
            )

        def out_cp(k, slot, b):
            return pltpu.make_async_copy(
                vbuf.at[slot],
                out_hbm.at[b, pl.ds(base + k * _CR, _CR), :],
                out_sem.at[slot],
            )

        for k in range(_RING - 1):
            in_cp(k, k % _RING).start()
        for k in range(nchunk):
            slot = k % _RING
            in_cp(k, slot).wait()
            nxt = k + _RING - 1
            if nxt < nchunk:
                nslot = nxt % _RING
                if k >= 1:
                    for b in range(B):
                        out_cp(k - 1, nslot, b).wait()
                in_cp(nxt, nslot).start()
            for b in range(B):
                out_cp(k, slot, b).start()
        for k in range(max(0, nchunk - _RING), nchunk):
            for b in range(B):
                out_cp(k, k % _RING, b).wait()

    return pl.kernel(
        body,
        out_type=jax.ShapeDtypeStruct((B, S, H), dtype),
        mesh=mesh,
        scratch_types=[
            pltpu.VMEM((_RING, _CR, H), dtype),
            pltpu.SemaphoreType.DMA((_RING,)),
            pltpu.SemaphoreType.DMA((_RING,)),
        ],
    )


def kernel(x, pos_emb):
    B, S = x.shape
    N, H = pos_emb.shape
    return _make_sc_kernel(B, S, H, pos_emb.dtype)(pos_emb)
